# baseline (speedup 1.0000x reference)
import jax
import jax.numpy as jnp
from jax import lax
from jax.experimental import pallas as pl
from jax.experimental.pallas import tpu as pltpu
from jax.experimental.pallas import tpu_sc as plsc

_N = 100000
_E = 6400000
_NC = 2
_NS = 16
_NW = _NC * _NS
_PER_W = _E // _NW
_W = 400
_NWIN = _PER_W // _W
_GROUPS = _W // 16
_RCHUNK = 400


def _edge_loss_body(pv_hbm, gv_hbm, eid_hbm, egt_hbm, out_hbm,
                    psp, gsp,
                    i0a, i0b, i0c, i0d, i1a, i1b, i1c, i1d,
                    r0a, r0b, r0c, r0d, r1a, r1b, r1c, r1d,
                    acc_v, isem, gsem):
    c = lax.axis_index("c")
    s = lax.axis_index("s")
    wid = s * _NC + c

    idx_bufs = ((i0a, i0b, i0c, i0d), (i1a, i1b, i1c, i1d))
    row_bufs = ((r0a, r0b, r0c, r0d), (r1a, r1b, r1c, r1d))

    def stage_chunk(t, _):
        j = s + t * _NS

        @pl.when(j < _N // _RCHUNK)
        def _do():
            r0 = j * _RCHUNK
            pltpu.sync_copy(pv_hbm.at[pl.ds(r0, _RCHUNK)], r0a)
            pltpu.sync_copy(r0a, psp.at[pl.ds(r0, _RCHUNK)])
            pltpu.sync_copy(gv_hbm.at[pl.ds(r0, _RCHUNK)], r0a)
            pltpu.sync_copy(r0a, gsp.at[pl.ds(r0, _RCHUNK)])

        return 0

    lax.fori_loop(0, (_N // _RCHUNK + _NS - 1) // _NS, stage_chunk, 0)

    plsc.subcore_barrier()

    base0 = wid * _PER_W

    def idx_copies(i, buf):
        base = base0 + i * _W
        ib = idx_bufs[buf]
        return [
            pltpu.make_async_copy(eid_hbm.at[pl.ds(base, _W)], ib[0], isem),
            pltpu.make_async_copy(eid_hbm.at[pl.ds(_E + base, _W)], ib[1], isem),
            pltpu.make_async_copy(egt_hbm.at[pl.ds(base, _W)], ib[2], isem),
            pltpu.make_async_copy(egt_hbm.at[pl.ds(_E + base, _W)], ib[3], isem),
        ]

    def gather_copies(buf):
        ib = idx_bufs[buf]
        rb = row_bufs[buf]
        return [
            pltpu.make_async_copy(psp.at[ib[0]], rb[0], gsem),
            pltpu.make_async_copy(psp.at[ib[1]], rb[1], gsem),
            pltpu.make_async_copy(gsp.at[ib[2]], rb[2], gsem),
            pltpu.make_async_copy(gsp.at[ib[3]], rb[3], gsem),
        ]

    def fire_idx(i, buf):
        for cp in idx_copies(i, buf):
            cp.start()

    def wait_idx(i, buf):
        for cp in idx_copies(i, buf):
            cp.wait()

    def fire_gathers(buf):
        for cp in gather_copies(buf):
            cp.start()

    def wait_gathers(buf):
        for cp in gather_copies(buf):
            cp.wait()

    lanes = lax.iota(jnp.int32, 16)
    c0 = jnp.zeros((16,), jnp.int32)
    c1 = c0 + 1
    c2 = c0 + 2

    def compute(buf, acc):
        pa, pb, ga, gb = row_bufs[buf]

        def group(j, a):
            e = j * 16 + lanes
            ax = plsc.load_gather(pa, [e, c0])
            ay = plsc.load_gather(pa, [e, c1])
            az = plsc.load_gather(pa, [e, c2])
            bx = plsc.load_gather(pb, [e, c0])
            by = plsc.load_gather(pb, [e, c1])
            bz = plsc.load_gather(pb, [e, c2])
            dx = ax - bx
            dy = ay - by
            dz = az - bz
            lp = dx * dx + dy * dy + dz * dz
            cx = plsc.load_gather(ga, [e, c0])
            cy = plsc.load_gather(ga, [e, c1])
            cz = plsc.load_gather(ga, [e, c2])
            ex = plsc.load_gather(gb, [e, c0])
            ey = plsc.load_gather(gb, [e, c1])
            ez = plsc.load_gather(gb, [e, c2])
            fx = cx - ex
            fy = cy - ey
            fz = cz - ez
            lg = fx * fx + fy * fy + fz * fz
            return a + jnp.abs(lp / lg - 1.0)

        return lax.fori_loop(0, _GROUPS, group, acc)

    fire_idx(0, 0)
    wait_idx(0, 0)
    fire_gathers(0)
    fire_idx(1, 1)

    def body(k, acc):
        i = k * 2
        wait_idx(i + 1, 1)
        wait_gathers(0)
        fire_gathers(1)
        nxt = jnp.minimum(i + 2, _NWIN - 2)
        fire_idx(nxt, 0)
        acc = compute(0, acc)
        wait_idx(nxt, 0)
        wait_gathers(1)
        fire_gathers(0)
        acc = compute(1, acc)
        nxt2 = jnp.minimum(i + 3, _NWIN - 1)
        fire_idx(nxt2, 1)
        return acc

    acc = lax.fori_loop(0, _NWIN // 2, body, jnp.zeros((16,), jnp.float32))
    wait_gathers(0)
    wait_idx(_NWIN - 1, 1)

    acc_v[...] = acc
    pltpu.sync_copy(acc_v, out_hbm.at[wid])


_edge_loss = pl.kernel(
    _edge_loss_body,
    out_type=jax.ShapeDtypeStruct((_NW, 16), jnp.float32),
    mesh=plsc.VectorSubcoreMesh(core_axis_name="c", subcore_axis_name="s"),
    compiler_params=pltpu.CompilerParams(use_tc_tiling_on_sc=False,
                                         needs_layout_passes=False),
    scratch_types=(
        [pltpu.VMEM_SHARED((_N, 6), jnp.float32)] * 2
        + [pltpu.VMEM((_W,), jnp.int32)] * 8
        + [pltpu.VMEM((_W, 6), jnp.float32)] * 8
        + [pltpu.VMEM((16,), jnp.float32)]
        + [pltpu.SemaphoreType.DMA] * 2
    ),
)


@jax.jit
def kernel(pred_v, edge_index_id, gt_v, edge_index_gt):
    if pred_v.ndim > 2:
        pred_v = pred_v.reshape((-1, 3))
        gt_v = gt_v.reshape((-1, 3))
    zpad = jnp.zeros((pred_v.shape[0], 3), jnp.float32)
    pv = jnp.concatenate([pred_v.astype(jnp.float32), zpad], axis=1)
    gv = jnp.concatenate([gt_v.astype(jnp.float32), zpad], axis=1)
    eid = edge_index_id.astype(jnp.int32).ravel()
    egt = edge_index_gt.astype(jnp.int32).ravel()
    partials = _edge_loss(pv, gv, eid, egt)
    return jnp.sum(partials) / jnp.float32(_E)


# speedup vs baseline: 1.1911x; 1.1911x over previous
import jax
import jax.numpy as jnp
from jax import lax
from jax.experimental import pallas as pl
from jax.experimental.pallas import tpu as pltpu
from jax.experimental.pallas import tpu_sc as plsc

_N = 100000
_E = 6400000
_NC = 2
_NS = 16
_NW = _NC * _NS
_PER_W = _E // _NW
_W = 800
_NWIN = _PER_W // _W
_GROUPS = _W // 16
_RCHUNK = 1000


def _edge_loss_body(pv_hbm, gv_hbm, eid_hbm, egt_hbm, out_hbm,
                    psp, gsp,
                    i0a, i0b, i0c, i0d, i1a, i1b, i1c, i1d,
                    r0a, r0b, r0c, r0d, r1a, r1b, r1c, r1d,
                    stage_v, acc_v, isem, gsem):
    c = lax.axis_index("c")
    s = lax.axis_index("s")
    wid = s * _NC + c

    idx_bufs = ((i0a, i0b, i0c, i0d), (i1a, i1b, i1c, i1d))
    row_bufs = ((r0a, r0b, r0c, r0d), (r1a, r1b, r1c, r1d))

    def stage_chunk(t, _):
        j = s + t * _NS

        @pl.when(j < _N // _RCHUNK)
        def _do():
            r0 = j * _RCHUNK
            pltpu.sync_copy(pv_hbm.at[pl.ds(r0, _RCHUNK)], stage_v)
            pltpu.sync_copy(stage_v, psp.at[pl.ds(r0, _RCHUNK)])
            pltpu.sync_copy(gv_hbm.at[pl.ds(r0, _RCHUNK)], stage_v)
            pltpu.sync_copy(stage_v, gsp.at[pl.ds(r0, _RCHUNK)])

        return 0

    lax.fori_loop(0, (_N // _RCHUNK + _NS - 1) // _NS, stage_chunk, 0)

    plsc.subcore_barrier()

    base0 = wid * _PER_W

    def idx_copies(i, buf):
        base = base0 + i * _W
        ib = idx_bufs[buf]
        return [
            pltpu.make_async_copy(eid_hbm.at[0, pl.ds(base, _W)], ib[0], isem),
            pltpu.make_async_copy(eid_hbm.at[1, pl.ds(base, _W)], ib[1], isem),
            pltpu.make_async_copy(egt_hbm.at[0, pl.ds(base, _W)], ib[2], isem),
            pltpu.make_async_copy(egt_hbm.at[1, pl.ds(base, _W)], ib[3], isem),
        ]

    def gather_copies(buf):
        ib = idx_bufs[buf]
        rb = row_bufs[buf]
        return [
            pltpu.make_async_copy(psp.at[ib[0]], rb[0], gsem),
            pltpu.make_async_copy(psp.at[ib[1]], rb[1], gsem),
            pltpu.make_async_copy(gsp.at[ib[2]], rb[2], gsem),
            pltpu.make_async_copy(gsp.at[ib[3]], rb[3], gsem),
        ]

    def fire_idx(i, buf):
        for cp in idx_copies(i, buf):
            cp.start()

    def wait_idx(i, buf):
        for cp in idx_copies(i, buf):
            cp.wait()

    def fire_gathers(buf):
        for cp in gather_copies(buf):
            cp.start()

    def wait_gathers(buf):
        for cp in gather_copies(buf):
            cp.wait()

    lanes = lax.iota(jnp.int32, 16)
    c0 = jnp.zeros((16,), jnp.int32)
    c1 = c0 + 1
    c2 = c0 + 2

    def compute(buf, acc):
        pa, pb, ga, gb = row_bufs[buf]

        def group(j, a):
            e = j * 16 + lanes
            ax = plsc.load_gather(pa, [e, c0])
            ay = plsc.load_gather(pa, [e, c1])
            az = plsc.load_gather(pa, [e, c2])
            bx = plsc.load_gather(pb, [e, c0])
            by = plsc.load_gather(pb, [e, c1])
            bz = plsc.load_gather(pb, [e, c2])
            dx = ax - bx
            dy = ay - by
            dz = az - bz
            lp = dx * dx + dy * dy + dz * dz
            cx = plsc.load_gather(ga, [e, c0])
            cy = plsc.load_gather(ga, [e, c1])
            cz = plsc.load_gather(ga, [e, c2])
            ex = plsc.load_gather(gb, [e, c0])
            ey = plsc.load_gather(gb, [e, c1])
            ez = plsc.load_gather(gb, [e, c2])
            fx = cx - ex
            fy = cy - ey
            fz = cz - ez
            lg = fx * fx + fy * fy + fz * fz
            return a + jnp.abs(lp / lg - 1.0)

        return lax.fori_loop(0, _GROUPS, group, acc)

    fire_idx(0, 0)
    wait_idx(0, 0)
    fire_gathers(0)
    fire_idx(1, 1)

    def body(k, acc):
        i = k * 2
        wait_idx(i + 1, 1)
        wait_gathers(0)
        fire_gathers(1)
        nxt = jnp.minimum(i + 2, _NWIN - 2)
        fire_idx(nxt, 0)
        acc = compute(0, acc)
        wait_idx(nxt, 0)
        wait_gathers(1)
        fire_gathers(0)
        acc = compute(1, acc)
        nxt2 = jnp.minimum(i + 3, _NWIN - 1)
        fire_idx(nxt2, 1)
        return acc

    acc = lax.fori_loop(0, _NWIN // 2, body, jnp.zeros((16,), jnp.float32))
    wait_gathers(0)
    wait_idx(_NWIN - 1, 1)

    acc_v[...] = acc
    pltpu.sync_copy(acc_v, out_hbm.at[wid])


_edge_loss = pl.kernel(
    _edge_loss_body,
    out_type=jax.ShapeDtypeStruct((_NW, 16), jnp.float32),
    mesh=plsc.VectorSubcoreMesh(core_axis_name="c", subcore_axis_name="s"),
    compiler_params=pltpu.CompilerParams(use_tc_tiling_on_sc=False,
                                         needs_layout_passes=False),
    scratch_types=(
        [pltpu.VMEM_SHARED((_N, 3), jnp.float32)] * 2
        + [pltpu.VMEM((_W,), jnp.int32)] * 8
        + [pltpu.VMEM((_W, 3), jnp.float32)] * 8
        + [pltpu.VMEM((_RCHUNK, 3), jnp.float32)]
        + [pltpu.VMEM((16,), jnp.float32)]
        + [pltpu.SemaphoreType.DMA] * 2
    ),
)


@jax.jit
def kernel(pred_v, edge_index_id, gt_v, edge_index_gt):
    if pred_v.ndim > 2:
        pred_v = pred_v.reshape((-1, 3))
        gt_v = gt_v.reshape((-1, 3))
    pv = pred_v.astype(jnp.float32)
    gv = gt_v.astype(jnp.float32)
    eid = edge_index_id.astype(jnp.int32)
    egt = edge_index_gt.astype(jnp.int32)
    partials = _edge_loss(pv, gv, eid, egt)
    return jnp.sum(partials) / jnp.float32(_E)


# speedup vs baseline: 1.2249x; 1.0284x over previous
import jax
import jax.numpy as jnp
from jax import lax
from jax.experimental import pallas as pl
from jax.experimental.pallas import tpu as pltpu
from jax.experimental.pallas import tpu_sc as plsc

_N = 100000
_E = 6400000
_NC = 2
_NS = 16
_NW = _NC * _NS
_PER_W = _E // _NW
_W = 1000
_NWIN = _PER_W // _W
_GROUPS = _W // 16
_RCHUNK = 1000


def _edge_loss_body(pv_hbm, gv_hbm, eid_hbm, egt_hbm, out_hbm,
                    psp, gsp,
                    i0a, i0b, i0c, i0d, i1a, i1b, i1c, i1d,
                    r0a, r0b, r0c, r0d, r1a, r1b, r1c, r1d,
                    stage_v, acc_v, isem, gsem):
    c = lax.axis_index("c")
    s = lax.axis_index("s")
    wid = s * _NC + c

    idx_bufs = ((i0a, i0b, i0c, i0d), (i1a, i1b, i1c, i1d))
    row_bufs = ((r0a, r0b, r0c, r0d), (r1a, r1b, r1c, r1d))

    def stage_chunk(t, _):
        j = s + t * _NS

        @pl.when(j < _N // _RCHUNK)
        def _do():
            r0 = j * _RCHUNK
            pltpu.sync_copy(pv_hbm.at[pl.ds(r0, _RCHUNK)], stage_v)
            pltpu.sync_copy(stage_v, psp.at[pl.ds(r0, _RCHUNK)])
            pltpu.sync_copy(gv_hbm.at[pl.ds(r0, _RCHUNK)], stage_v)
            pltpu.sync_copy(stage_v, gsp.at[pl.ds(r0, _RCHUNK)])

        return 0

    lax.fori_loop(0, (_N // _RCHUNK + _NS - 1) // _NS, stage_chunk, 0)

    plsc.subcore_barrier()

    base0 = wid * _PER_W

    def idx_copies(i, buf):
        base = base0 + i * _W
        ib = idx_bufs[buf]
        return [
            pltpu.make_async_copy(eid_hbm.at[0, pl.ds(base, _W)], ib[0], isem),
            pltpu.make_async_copy(eid_hbm.at[1, pl.ds(base, _W)], ib[1], isem),
            pltpu.make_async_copy(egt_hbm.at[0, pl.ds(base, _W)], ib[2], isem),
            pltpu.make_async_copy(egt_hbm.at[1, pl.ds(base, _W)], ib[3], isem),
        ]

    def gather_copies(buf):
        ib = idx_bufs[buf]
        rb = row_bufs[buf]
        return [
            pltpu.make_async_copy(psp.at[ib[0]], rb[0], gsem),
            pltpu.make_async_copy(psp.at[ib[1]], rb[1], gsem),
            pltpu.make_async_copy(gsp.at[ib[2]], rb[2], gsem),
            pltpu.make_async_copy(gsp.at[ib[3]], rb[3], gsem),
        ]

    def fire_idx(i, buf):
        for cp in idx_copies(i, buf):
            cp.start()

    def wait_idx(i, buf):
        for cp in idx_copies(i, buf):
            cp.wait()

    def fire_gathers(buf):
        for cp in gather_copies(buf):
            cp.start()

    def wait_gathers(buf):
        for cp in gather_copies(buf):
            cp.wait()

    lanes = lax.iota(jnp.int32, 16)
    c0 = jnp.zeros((16,), jnp.int32)
    c1 = c0 + 1
    c2 = c0 + 2

    def compute(buf, acc):
        pa, pb, ga, gb = row_bufs[buf]

        def group(j, a):
            e = j * 16 + lanes
            ax = plsc.load_gather(pa, [e, c0])
            ay = plsc.load_gather(pa, [e, c1])
            az = plsc.load_gather(pa, [e, c2])
            bx = plsc.load_gather(pb, [e, c0])
            by = plsc.load_gather(pb, [e, c1])
            bz = plsc.load_gather(pb, [e, c2])
            dx = ax - bx
            dy = ay - by
            dz = az - bz
            lp = dx * dx + dy * dy + dz * dz
            cx = plsc.load_gather(ga, [e, c0])
            cy = plsc.load_gather(ga, [e, c1])
            cz = plsc.load_gather(ga, [e, c2])
            ex = plsc.load_gather(gb, [e, c0])
            ey = plsc.load_gather(gb, [e, c1])
            ez = plsc.load_gather(gb, [e, c2])
            fx = cx - ex
            fy = cy - ey
            fz = cz - ez
            lg = fx * fx + fy * fy + fz * fz
            return a + jnp.abs(lp / lg - 1.0)

        return lax.fori_loop(0, _GROUPS, group, acc)

    fire_idx(0, 0)
    wait_idx(0, 0)
    fire_gathers(0)
    fire_idx(1, 1)

    def body(k, acc):
        i = k * 2
        wait_idx(i + 1, 1)
        wait_gathers(0)
        fire_gathers(1)
        nxt = jnp.minimum(i + 2, _NWIN - 2)
        fire_idx(nxt, 0)
        acc = compute(0, acc)
        wait_idx(nxt, 0)
        wait_gathers(1)
        fire_gathers(0)
        acc = compute(1, acc)
        nxt2 = jnp.minimum(i + 3, _NWIN - 1)
        fire_idx(nxt2, 1)
        return acc

    acc = lax.fori_loop(0, _NWIN // 2, body, jnp.zeros((16,), jnp.float32))
    wait_gathers(0)
    wait_idx(_NWIN - 1, 1)

    acc_v[...] = acc
    pltpu.sync_copy(acc_v, out_hbm.at[wid])


_edge_loss = pl.kernel(
    _edge_loss_body,
    out_type=jax.ShapeDtypeStruct((_NW, 16), jnp.float32),
    mesh=plsc.VectorSubcoreMesh(core_axis_name="c", subcore_axis_name="s"),
    compiler_params=pltpu.CompilerParams(use_tc_tiling_on_sc=False,
                                         needs_layout_passes=False),
    scratch_types=(
        [pltpu.VMEM_SHARED((_N, 3), jnp.float32)] * 2
        + [pltpu.VMEM((_W,), jnp.int32)] * 8
        + [pltpu.VMEM((_W, 3), jnp.float32)] * 8
        + [pltpu.VMEM((_RCHUNK, 3), jnp.float32)]
        + [pltpu.VMEM((16,), jnp.float32)]
        + [pltpu.SemaphoreType.DMA] * 2
    ),
)


@jax.jit
def kernel(pred_v, edge_index_id, gt_v, edge_index_gt):
    if pred_v.ndim > 2:
        pred_v = pred_v.reshape((-1, 3))
        gt_v = gt_v.reshape((-1, 3))
    pv = pred_v.astype(jnp.float32)
    gv = gt_v.astype(jnp.float32)
    eid = edge_index_id.astype(jnp.int32)
    egt = edge_index_gt.astype(jnp.int32)
    partials = _edge_loss(pv, gv, eid, egt)
    return jnp.sum(partials) / jnp.float32(_E)
